# baseline (speedup 1.0000x reference)
import jax
import jax.numpy as jnp
from jax.experimental import pallas as pl

B, S, D, DFF, E, R = 2, 2048, 1024, 4096, 8, 32
ER = E * R
N1 = DFF + ER + E
N2 = D + ER
NC = ER + E
SCALING = 1.0 / 32.0
TM = 256


def _dot(a, b, dims):
    return jax.lax.dot_general(a, b, (dims, ((), ())),
                               preferred_element_type=jnp.float32)


def _moe_kernel(x_ref, w1c_ref, b1c_ref, bc1_ref, w2c_ref, b2c_ref, bc2_ref,
                out_ref, routing_ref, ec_ref):
    bf = jnp.bfloat16
    xb = x_ref[...].astype(bf)

    p1 = _dot(xb, w1c_ref[...], (((1,), (1,)))) + b1c_ref[...]
    o1 = p1[:, :DFF]
    h1 = p1[:, DFF:DFF + ER]
    logits = p1[:, DFF + ER:]

    m = jnp.max(logits, axis=-1, keepdims=True)
    ex = jnp.exp(logits - m)
    r = ex / jnp.sum(ex, axis=-1, keepdims=True)
    routing_ref[...] = r
    iot = jax.lax.broadcasted_iota(jnp.int32, (TM, E), 1)
    rmax = jnp.max(r, axis=-1, keepdims=True)
    amin = jnp.min(jnp.where(r == rmax, iot, E), axis=-1, keepdims=True)
    ec_ref[...] = (iot == amin).astype(jnp.float32)

    erow = jax.lax.broadcasted_iota(jnp.int32, (E, ER), 0)
    ecol = jax.lax.broadcasted_iota(jnp.int32, (E, ER), 1)
    expand = (erow == ecol // R).astype(bf)
    r16 = r.astype(bf)
    rw = _dot(r16, expand, (((1,), (0,))))

    h1c = jnp.concatenate([(h1 * rw).astype(bf), r16], axis=1)
    l1 = _dot(h1c, bc1_ref[...], (((1,), (0,))))
    down = o1 + SCALING * l1
    a = (0.5 * down * (1.0 + jax.lax.erf(down * 0.7071067811865476))).astype(bf)

    p2 = _dot(a, w2c_ref[...], (((1,), (1,)))) + b2c_ref[...]
    o2 = p2[:, :D]
    h2 = p2[:, D:]

    h2c = jnp.concatenate([(h2 * rw).astype(bf), r16], axis=1)
    l2 = _dot(h2c, bc2_ref[...], (((1,), (0,))))
    out_ref[...] = o2 + SCALING * l2


@jax.jit
def kernel(x, Wr, br, W1, b1, W2, b2, A_down, bA_down, B_down, bB_down,
           A_up, bA_up, B_up, bB_up):
    bf = jnp.bfloat16
    T = B * S
    x2 = x.reshape(T, D)
    w1c = jnp.concatenate([W1, A_down.reshape(ER, D), Wr], axis=0).astype(bf)
    b1c = jnp.concatenate([b1, bA_down.reshape(ER), br]).reshape(1, N1)
    bc1 = jnp.concatenate([B_down.transpose(0, 2, 1).reshape(ER, DFF),
                           bB_down], axis=0).astype(bf)
    w2c = jnp.concatenate([W2, A_up.reshape(ER, DFF)], axis=0).astype(bf)
    b2c = jnp.concatenate([b2, bA_up.reshape(ER)]).reshape(1, N2)
    bc2 = jnp.concatenate([B_up.transpose(0, 2, 1).reshape(ER, D),
                           bB_up], axis=0).astype(bf)

    grid = (T // TM,)
    tok = lambda i: (i, 0)
    fixed = lambda i: (0, 0)

    out2, routing2, ec2 = pl.pallas_call(
        _moe_kernel,
        grid=grid,
        in_specs=[
            pl.BlockSpec((TM, D), tok),
            pl.BlockSpec((N1, D), fixed),
            pl.BlockSpec((1, N1), fixed),
            pl.BlockSpec((NC, DFF), fixed),
            pl.BlockSpec((N2, DFF), fixed),
            pl.BlockSpec((1, N2), fixed),
            pl.BlockSpec((NC, D), fixed),
        ],
        out_specs=[
            pl.BlockSpec((TM, D), tok),
            pl.BlockSpec((TM, E), tok),
            pl.BlockSpec((TM, E), tok),
        ],
        out_shape=[
            jax.ShapeDtypeStruct((T, D), jnp.float32),
            jax.ShapeDtypeStruct((T, E), jnp.float32),
            jax.ShapeDtypeStruct((T, E), jnp.float32),
        ],
    )(x2, w1c, b1c, bc1, w2c, b2c, bc2)

    return (out2.reshape(B, S, D), routing2.reshape(B, S, E),
            ec2.reshape(B, S, E))


# speedup vs baseline: 1.2521x; 1.2521x over previous
import jax
import jax.numpy as jnp
from jax.experimental import pallas as pl
from jax.experimental.pallas import tpu as pltpu

B, S, D, DFF, E, R = 2, 2048, 1024, 4096, 8, 32
ER = E * R
SCALING = 1.0 / 32.0
TM = 256
P = 4
W1C, W2C = DFF // P, D // P
AC = ER // P


def _dot(a, b, dims):
    return jax.lax.dot_general(a, b, (dims, ((), ())),
                               preferred_element_type=jnp.float32)


def _moe_kernel(x_ref, wr_ref, br_ref, w1_ref, b1_ref, w2_ref, b2_ref,
                adn_ref, badn_ref, bdn_ref, bbdn_ref,
                aup_ref, baup_ref, bup_ref, bbup_ref,
                out_ref, routing_ref, ec_ref,
                w1s, w2s, adns, aups):
    bf = jnp.bfloat16
    i = pl.program_id(0)

    @pl.when(i < P)
    def _prep():
        w1s[pl.ds(i * W1C, W1C), :] = w1_ref[...].astype(bf)
        w2s[pl.ds(i * W2C, W2C), :] = w2_ref[...].astype(bf)
        adns[pl.ds(i * AC, AC), :] = adn_ref[...].astype(bf)
        aups[pl.ds(i * AC, AC), :] = aup_ref[...].astype(bf)

    @pl.when(i >= P)
    def _main():
        xb = x_ref[...].astype(bf)

        logits = _dot(xb, wr_ref[...].astype(bf), (((1,), (1,)))) + br_ref[...]
        m = jnp.max(logits, axis=-1, keepdims=True)
        ex = jnp.exp(logits - m)
        r = ex / jnp.sum(ex, axis=-1, keepdims=True)
        routing_ref[...] = r
        iot = jax.lax.broadcasted_iota(jnp.int32, (TM, E), 1)
        rmax = jnp.max(r, axis=-1, keepdims=True)
        amin = jnp.min(jnp.where(r == rmax, iot, E), axis=-1, keepdims=True)
        ec_ref[...] = (iot == amin).astype(jnp.float32)

        erow = jax.lax.broadcasted_iota(jnp.int32, (E, ER), 0)
        ecol = jax.lax.broadcasted_iota(jnp.int32, (E, ER), 1)
        expand = (erow == ecol // R).astype(bf)
        r16 = r.astype(bf)
        rw = _dot(r16, expand, (((1,), (0,))))

        h1 = _dot(xb, adns[...], (((1,), (1,)))) + badn_ref[...]
        h1w = (h1 * rw).astype(bf)
        o1 = _dot(xb, w1s[...], (((1,), (1,))))
        l1 = _dot(h1w, bdn_ref[...], (((1,), (0,)))) \
            + _dot(r16, bbdn_ref[...], (((1,), (0,))))
        down = o1 + b1_ref[...] + SCALING * l1
        a = (0.5 * down
             * (1.0 + jax.lax.erf(down * 0.7071067811865476))).astype(bf)

        h2 = _dot(a, aups[...], (((1,), (1,)))) + baup_ref[...]
        h2w = (h2 * rw).astype(bf)
        o2 = _dot(a, w2s[...], (((1,), (1,))))
        l2 = _dot(h2w, bup_ref[...], (((1,), (0,)))) \
            + _dot(r16, bbup_ref[...], (((1,), (0,))))
        out_ref[...] = o2 + b2_ref[...] + SCALING * l2


@jax.jit
def kernel(x, Wr, br, W1, b1, W2, b2, A_down, bA_down, B_down, bB_down,
           A_up, bA_up, B_up, bB_up):
    bf = jnp.bfloat16
    T = B * S
    x2 = x.reshape(T, D)
    bdn = B_down.transpose(0, 2, 1).reshape(ER, DFF).astype(bf)
    bup = B_up.transpose(0, 2, 1).reshape(ER, D).astype(bf)

    grid = (P + T // TM,)
    tok = lambda i: (jnp.maximum(i - P, 0), 0)
    fixed = lambda i: (0, 0)
    chunk = lambda i: (jnp.minimum(i, P - 1), 0)

    out2, routing2, ec2 = pl.pallas_call(
        _moe_kernel,
        grid=grid,
        in_specs=[
            pl.BlockSpec((TM, D), tok),
            pl.BlockSpec((E, D), fixed),
            pl.BlockSpec((1, E), fixed),
            pl.BlockSpec((W1C, D), chunk),
            pl.BlockSpec((1, DFF), fixed),
            pl.BlockSpec((W2C, DFF), chunk),
            pl.BlockSpec((1, D), fixed),
            pl.BlockSpec((AC, D), chunk),
            pl.BlockSpec((1, ER), fixed),
            pl.BlockSpec((ER, DFF), fixed),
            pl.BlockSpec((E, DFF), fixed),
            pl.BlockSpec((AC, DFF), chunk),
            pl.BlockSpec((1, ER), fixed),
            pl.BlockSpec((ER, D), fixed),
            pl.BlockSpec((E, D), fixed),
        ],
        out_specs=[
            pl.BlockSpec((TM, D), tok),
            pl.BlockSpec((TM, E), tok),
            pl.BlockSpec((TM, E), tok),
        ],
        out_shape=[
            jax.ShapeDtypeStruct((T, D), jnp.float32),
            jax.ShapeDtypeStruct((T, E), jnp.float32),
            jax.ShapeDtypeStruct((T, E), jnp.float32),
        ],
        scratch_shapes=[
            pltpu.VMEM((DFF, D), bf),
            pltpu.VMEM((D, DFF), bf),
            pltpu.VMEM((ER, D), bf),
            pltpu.VMEM((ER, DFF), bf),
        ],
    )(x2, Wr, br.reshape(1, E), W1, b1.reshape(1, DFF), W2, b2.reshape(1, D),
      A_down.reshape(ER, D), bA_down.reshape(1, ER), bdn,
      bB_down.astype(bf), A_up.reshape(ER, DFF), bA_up.reshape(1, ER),
      bup, bB_up.astype(bf))

    return (out2.reshape(B, S, D), routing2.reshape(B, S, E),
            ec2.reshape(B, S, E))


# speedup vs baseline: 1.3039x; 1.0414x over previous
import jax
import jax.numpy as jnp
from jax.experimental import pallas as pl
from jax.experimental.pallas import tpu as pltpu

B, S, D, DFF, E, R = 2, 2048, 1024, 4096, 8, 32
ER = E * R
SCALING = 1.0 / 32.0
TM = 512
P = 4
W1C, W2C = DFF // P, D // P
AC = ER // P


def _dot(a, b, dims):
    return jax.lax.dot_general(a, b, (dims, ((), ())),
                               preferred_element_type=jnp.float32)


def _moe_kernel(x_ref, wr_ref, br_ref, w1_ref, b1_ref, w2_ref, b2_ref,
                adn_ref, badn_ref, bdn_ref, bbdn_ref,
                aup_ref, baup_ref, bup_ref, bbup_ref,
                out_ref, routing_ref, ec_ref,
                w1s, w2s, adns, aups):
    bf = jnp.bfloat16
    i = pl.program_id(0)

    @pl.when(i < P)
    def _prep():
        w1s[pl.ds(i * W1C, W1C), :] = w1_ref[...].astype(bf)
        w2s[pl.ds(i * W2C, W2C), :] = w2_ref[...].astype(bf)
        adns[pl.ds(i * AC, AC), :] = adn_ref[...].astype(bf)
        aups[pl.ds(i * AC, AC), :] = aup_ref[...].astype(bf)

    @pl.when(i >= P)
    def _main():
        xb = x_ref[...].astype(bf)

        logits = _dot(xb, wr_ref[...].astype(bf), (((1,), (1,)))) + br_ref[...]
        m = jnp.max(logits, axis=-1, keepdims=True)
        ex = jnp.exp(logits - m)
        r = ex / jnp.sum(ex, axis=-1, keepdims=True)
        routing_ref[...] = r
        iot = jax.lax.broadcasted_iota(jnp.int32, (TM, E), 1)
        rmax = jnp.max(r, axis=-1, keepdims=True)
        amin = jnp.min(jnp.where(r == rmax, iot, E), axis=-1, keepdims=True)
        ec_ref[...] = (iot == amin).astype(jnp.float32)

        erow = jax.lax.broadcasted_iota(jnp.int32, (E, ER), 0)
        ecol = jax.lax.broadcasted_iota(jnp.int32, (E, ER), 1)
        expand = (erow == ecol // R).astype(bf)
        r16 = r.astype(bf)
        rw = _dot(r16, expand, (((1,), (0,))))

        h1 = _dot(xb, adns[...], (((1,), (1,)))) + badn_ref[...]
        h1w = (h1 * rw).astype(bf)
        o1 = _dot(xb, w1s[...], (((1,), (1,))))
        l1 = _dot(h1w, bdn_ref[...], (((1,), (0,))))
        down = o1 + b1_ref[...] + SCALING * l1
        a = (0.5 * down
             * (1.0 + jax.lax.erf(down * 0.7071067811865476))).astype(bf)

        h2 = _dot(a, aups[...], (((1,), (1,)))) + baup_ref[...]
        h2w = (h2 * rw).astype(bf)
        o2 = _dot(a, w2s[...], (((1,), (1,))))
        l2 = _dot(h2w, bup_ref[...], (((1,), (0,))))
        out_ref[...] = o2 + b2_ref[...] + SCALING * l2


@jax.jit
def kernel(x, Wr, br, W1, b1, W2, b2, A_down, bA_down, B_down, bB_down,
           A_up, bA_up, B_up, bB_up):
    bf = jnp.bfloat16
    T = B * S
    x2 = x.reshape(T, D)
    bdn = B_down.transpose(0, 2, 1).reshape(ER, DFF).astype(bf)
    bup = B_up.transpose(0, 2, 1).reshape(ER, D).astype(bf)

    grid = (P + T // TM,)
    tok = lambda i: (jnp.maximum(i - P, 0), 0)
    fixed = lambda i: (0, 0)
    chunk = lambda i: (jnp.minimum(i, P - 1), 0)

    out2, routing2, ec2 = pl.pallas_call(
        _moe_kernel,
        grid=grid,
        in_specs=[
            pl.BlockSpec((TM, D), tok),
            pl.BlockSpec((E, D), fixed),
            pl.BlockSpec((1, E), fixed),
            pl.BlockSpec((W1C, D), chunk),
            pl.BlockSpec((1, DFF), fixed),
            pl.BlockSpec((W2C, DFF), chunk),
            pl.BlockSpec((1, D), fixed),
            pl.BlockSpec((AC, D), chunk),
            pl.BlockSpec((1, ER), fixed),
            pl.BlockSpec((ER, DFF), fixed),
            pl.BlockSpec((E, DFF), fixed),
            pl.BlockSpec((AC, DFF), chunk),
            pl.BlockSpec((1, ER), fixed),
            pl.BlockSpec((ER, D), fixed),
            pl.BlockSpec((E, D), fixed),
        ],
        out_specs=[
            pl.BlockSpec((TM, D), tok),
            pl.BlockSpec((TM, E), tok),
            pl.BlockSpec((TM, E), tok),
        ],
        out_shape=[
            jax.ShapeDtypeStruct((T, D), jnp.float32),
            jax.ShapeDtypeStruct((T, E), jnp.float32),
            jax.ShapeDtypeStruct((T, E), jnp.float32),
        ],
        scratch_shapes=[
            pltpu.VMEM((DFF, D), bf),
            pltpu.VMEM((D, DFF), bf),
            pltpu.VMEM((ER, D), bf),
            pltpu.VMEM((ER, DFF), bf),
        ],
    )(x2, Wr, br.reshape(1, E), W1, b1.reshape(1, DFF), W2, b2.reshape(1, D),
      A_down.reshape(ER, D), bA_down.reshape(1, ER), bdn,
      bB_down.astype(bf), A_up.reshape(ER, DFF), bA_up.reshape(1, ER),
      bup, bB_up.astype(bf))

    return (out2.reshape(B, S, D), routing2.reshape(B, S, E),
            ec2.reshape(B, S, E))
